# baseline (device time: 45684 ns/iter reference)
import contextlib
import os

import jax
import jax.numpy as jnp
from jax import lax
from jax.experimental import pallas as pl
from jax.experimental.pallas import tpu as pltpu

_SCOPES = os.environ.get("KERNEL_SCOPES") == "1"


def _scope(name):
    return jax.named_scope(name) if _SCOPES else contextlib.nullcontext()


def kernel(x):
    m, n = x.shape
    h = m // 2
    q = h // 2
    e = q // 2

    def body(x_ref, out_ref, abuf, bbuf,
             r1af, r1ak, r1bf, r1bk, r2a, r2b,
             send_sems, recv_sems):
        dx = lax.axis_index("x")
        dy = lax.axis_index("y")
        y_nbr = (dx, 1 - dy)
        x_nbr = (1 - dx, dy)

        with _scope("barrier"):
            barrier_sem = pltpu.get_barrier_semaphore()
            for nbr in (y_nbr, x_nbr):
                pl.semaphore_signal(
                    barrier_sem, inc=1,
                    device_id=nbr, device_id_type=pl.DeviceIdType.MESH,
                )
            pl.semaphore_wait(barrier_sem, 2)

        def exch(src, dst, slot, nbr):
            return pltpu.make_async_remote_copy(
                src_ref=src, dst_ref=dst,
                send_sem=send_sems.at[slot], recv_sem=recv_sems.at[slot],
                device_id=nbr, device_id_type=pl.DeviceIdType.MESH,
            )

        a_keep = dy * q
        a_fwd = a_keep + (1 - dx) * e
        a_own = a_keep + dx * e
        b_keep = dx * q
        b_fwd = b_keep + (1 - dy) * e
        b_own = b_keep + dy * e

        with _scope("cast_send"):
            afwd_src = (1 - dy) * q + (1 - dx) * e
            bfwd_src = (1 - dx) * q + (1 - dy) * e
            akeep_src = (1 - dy) * q + dx * e
            bkeep_src = (1 - dx) * q + dy * e
            abuf[pl.ds(afwd_src, e), :] = x_ref[
                pl.ds(afwd_src, e), :
            ].astype(jnp.bfloat16)
            a1f = exch(abuf.at[pl.ds(afwd_src, e)], r1af, 0, y_nbr)
            a1f.start()
            bbuf[pl.ds(bfwd_src, e), :] = x_ref[
                pl.ds(h + bfwd_src, e), :
            ].astype(jnp.bfloat16)
            b1f = exch(bbuf.at[pl.ds(bfwd_src, e)], r1bf, 1, x_nbr)
            b1f.start()
            abuf[pl.ds(akeep_src, e), :] = x_ref[
                pl.ds(akeep_src, e), :
            ].astype(jnp.bfloat16)
            a1k = exch(abuf.at[pl.ds(akeep_src, e)], r1ak, 2, y_nbr)
            a1k.start()
            bbuf[pl.ds(bkeep_src, e), :] = x_ref[
                pl.ds(h + bkeep_src, e), :
            ].astype(jnp.bfloat16)
            b1k = exch(bbuf.at[pl.ds(bkeep_src, e)], r1bk, 3, x_nbr)
            b1k.start()
        with _scope("wait_a1f"):
            a1f.wait()
        with _scope("add_afwd"):
            abuf[pl.ds(a_fwd, e), :] = (
                x_ref[pl.ds(a_fwd, e), :] + r1af[...].astype(jnp.float32)
            ).astype(jnp.bfloat16)
            a2 = exch(abuf.at[pl.ds(a_fwd, e)], r2a, 4, x_nbr)
            a2.start()

        with _scope("wait_b1f"):
            b1f.wait()
        with _scope("add_bfwd"):
            bbuf[pl.ds(b_fwd, e), :] = (
                x_ref[pl.ds(h + b_fwd, e), :] + r1bf[...].astype(jnp.float32)
            ).astype(jnp.bfloat16)
            b2 = exch(bbuf.at[pl.ds(b_fwd, e)], r2b, 5, y_nbr)
            b2.start()

        with _scope("wait_a1k_a2"):
            a1k.wait()
            a2.wait()
        with _scope("store_aown"):
            out_ref[pl.ds(a_own, e), :] = (
                x_ref[pl.ds(a_own, e), :]
                + (r1ak[...] + r2a[...]).astype(jnp.float32)
            ).astype(jnp.bfloat16)
            a3 = exch(out_ref.at[pl.ds(a_own, e)],
                      out_ref.at[pl.ds(a_own, e)], 6, x_nbr)
            a3.start()

        with _scope("wait_b1k_b2"):
            b1k.wait()
            b2.wait()
        with _scope("store_bown"):
            out_ref[pl.ds(h + b_own, e), :] = (
                x_ref[pl.ds(h + b_own, e), :]
                + (r1bk[...] + r2b[...]).astype(jnp.float32)
            ).astype(jnp.bfloat16)
            b3 = exch(out_ref.at[pl.ds(h + b_own, e)],
                      out_ref.at[pl.ds(h + b_own, e)], 7, y_nbr)
            b3.start()

        with _scope("start_ag_own"):
            a4a = exch(out_ref.at[pl.ds(a_own, e)],
                       out_ref.at[pl.ds(a_own, e)], 8, y_nbr)
            a4a.start()
            b4a = exch(out_ref.at[pl.ds(h + b_own, e)],
                       out_ref.at[pl.ds(h + b_own, e)], 9, x_nbr)
            b4a.start()

        with _scope("wait_a3"):
            a3.wait()
        with _scope("start_a4b"):
            a4b = exch(out_ref.at[pl.ds(a_keep + (1 - dx) * e, e)],
                       out_ref.at[pl.ds(a_keep + (1 - dx) * e, e)], 10, y_nbr)
            a4b.start()
        with _scope("wait_b3"):
            b3.wait()
        with _scope("start_b4b"):
            b4b = exch(out_ref.at[pl.ds(h + b_keep + (1 - dy) * e, e)],
                       out_ref.at[pl.ds(h + b_keep + (1 - dy) * e, e)],
                       11, x_nbr)
            b4b.start()

        with _scope("wait_tail"):
            a4a.wait()
            b4a.wait()
            a4b.wait()
            b4b.wait()

    return pl.pallas_call(
        body,
        out_shape=jax.ShapeDtypeStruct((m, n), jnp.bfloat16),
        in_specs=[pl.BlockSpec(memory_space=pltpu.VMEM)],
        out_specs=pl.BlockSpec(memory_space=pltpu.VMEM),
        scratch_shapes=[
            pltpu.VMEM((h, n), jnp.bfloat16),
            pltpu.VMEM((h, n), jnp.bfloat16),
            pltpu.VMEM((e, n), jnp.bfloat16),
            pltpu.VMEM((e, n), jnp.bfloat16),
            pltpu.VMEM((e, n), jnp.bfloat16),
            pltpu.VMEM((e, n), jnp.bfloat16),
            pltpu.VMEM((e, n), jnp.bfloat16),
            pltpu.VMEM((e, n), jnp.bfloat16),
            pltpu.SemaphoreType.DMA((12,)),
            pltpu.SemaphoreType.DMA((12,)),
        ],
        compiler_params=pltpu.CompilerParams(collective_id=0),
    )(x)
